# Initial kernel scaffold; baseline (speedup 1.0000x reference)
import jax
import jax.numpy as jnp
from jax.experimental import pallas as pl


def kernel(input, table):
    raise NotImplementedError("write your pallas kernel here")


# speedup vs baseline: 2.3678x; 2.3678x over previous
import functools

import jax
import jax.numpy as jnp
from jax import lax
from jax.experimental import pallas as pl
from jax.experimental.pallas import tpu as pltpu
from jax.experimental.pallas import tpu_sc as plsc

NC, NS, L = 2, 16, 16
NW = NC * NS
BATCH, HIST, D = 16384, 50, 128
B = BATCH * HIST
CB = 128
BPW = B // NW
NCHUNK = BPW // CB


def _body(xs_hbm, ys_hbm, zs_hbm, table_hbm, out_hbm,
          xs_v, ys_v, zs_v, idx_v, rows_v, sem):
    wid = lax.axis_index("s") * NC + lax.axis_index("c")
    base = wid * BPW

    def chunk_body(c, carry):
        row0 = base + c * CB
        pltpu.sync_copy(xs_hbm.at[pl.ds(row0, CB)], xs_v)
        pltpu.sync_copy(ys_hbm.at[pl.ds(row0, CB)], ys_v)
        pltpu.sync_copy(zs_hbm.at[pl.ds(row0, CB)], zs_v)

        def idx_body(j, carry2):
            s = pl.ds(j * L, L)
            idx_v[s] = (xs_v[s] * 16 + ys_v[s]) * 16 + zs_v[s]
            return carry2

        lax.fori_loop(0, CB // L, idx_body, 0)
        pltpu.async_copy(table_hbm.at[idx_v], rows_v, sem).wait()
        pltpu.sync_copy(rows_v, out_hbm.at[pl.ds(row0, CB)])
        return carry

    lax.fori_loop(0, NCHUNK, chunk_body, 0)


_gather = functools.partial(
    pl.kernel,
    out_type=jax.ShapeDtypeStruct((B, D), jnp.float32),
    mesh=plsc.VectorSubcoreMesh(core_axis_name="c", subcore_axis_name="s"),
    scratch_types=[
        pltpu.VMEM((CB,), jnp.int32),
        pltpu.VMEM((CB,), jnp.int32),
        pltpu.VMEM((CB,), jnp.int32),
        pltpu.VMEM((CB,), jnp.int32),
        pltpu.VMEM((CB, D), jnp.float32),
        pltpu.SemaphoreType.DMA,
    ],
)(_body)


@jax.jit
def kernel(input, table):
    flat = input.reshape(B, 3)
    xs = flat[:, 0].reshape(B)
    ys = flat[:, 1].reshape(B)
    zs = flat[:, 2].reshape(B)
    return _gather(xs, ys, zs, table).reshape(BATCH, HIST, D)


# speedup vs baseline: 3.1939x; 1.3489x over previous
import functools

import jax
import jax.numpy as jnp
from jax import lax
from jax.experimental import pallas as pl
from jax.experimental.pallas import tpu as pltpu
from jax.experimental.pallas import tpu_sc as plsc

NC, NS, L = 2, 16, 16
NW = NC * NS
BATCH, HIST, D = 16384, 50, 128
B = BATCH * HIST
CB = 128
BPW = B // NW
NCHUNK = BPW // CB
NBUF = 4
ROUNDS = NCHUNK // NBUF


def _body(xs_hbm, ys_hbm, zs_hbm, table_hbm, out_hbm,
          xs_v, ys_v, zs_v, idx_v, rows_v, *sems):
    csem = sems[0:NBUF]
    gsem = sems[NBUF:2 * NBUF]
    osem = sems[2 * NBUF:3 * NBUF]
    wid = lax.axis_index("s") * NC + lax.axis_index("c")
    base = wid * BPW

    def fire_coords(g, b):
        row0 = base + g * CB
        pltpu.async_copy(xs_hbm.at[pl.ds(row0, CB)], xs_v.at[b], csem[b])
        pltpu.async_copy(ys_hbm.at[pl.ds(row0, CB)], ys_v.at[b], csem[b])
        pltpu.async_copy(zs_hbm.at[pl.ds(row0, CB)], zs_v.at[b], csem[b])

    def wait_coords(b):
        for ref in (xs_v, ys_v, zs_v):
            pltpu.make_async_copy(xs_hbm.at[pl.ds(0, CB)], ref.at[b],
                                  csem[b]).wait()

    def compute_idx(b):
        for j in range(CB // L):
            s = slice(j * L, (j + 1) * L)
            idx_v[b, s] = (xs_v[b, s] * 16 + ys_v[b, s]) * 16 + zs_v[b, s]

    def fire_gather(b):
        pltpu.async_copy(table_hbm.at[idx_v.at[b]], rows_v.at[b], gsem[b])

    def wait_gather(b):
        pltpu.make_async_copy(table_hbm.at[idx_v.at[b]], rows_v.at[b],
                              gsem[b]).wait()

    def fire_out(g, b):
        pltpu.async_copy(rows_v.at[b], out_hbm.at[pl.ds(base + g * CB, CB)],
                         osem[b])

    def wait_out(b):
        pltpu.make_async_copy(out_hbm.at[pl.ds(base, CB)], rows_v.at[b],
                              osem[b]).wait()

    for b in range(NBUF):
        fire_coords(b, b)

    def round_body(r, carry):
        for b in range(NBUF):
            g = r * NBUF + b
            wait_coords(b)
            compute_idx(b)

            @pl.when(r > 0)
            def _():
                wait_out(b)

            fire_gather(b)
            pb = (b - 1) % NBUF
            if b > 0:
                wait_gather(pb)
                fire_out(g - 1, pb)
            else:
                @pl.when(r > 0)
                def _():
                    wait_gather(pb)
                    fire_out(g - 1, pb)

            @pl.when(r < ROUNDS - 1)
            def _():
                fire_coords(g + NBUF, b)
        return carry

    lax.fori_loop(0, ROUNDS, round_body, 0)

    bl = (NCHUNK - 1) % NBUF
    wait_gather(bl)
    pltpu.sync_copy(rows_v.at[bl], out_hbm.at[pl.ds(base + (NCHUNK - 1) * CB, CB)])
    for b in range(NBUF):
        if b != bl:
            wait_out(b)


_gather = functools.partial(
    pl.kernel,
    out_type=jax.ShapeDtypeStruct((B, D), jnp.float32),
    mesh=plsc.VectorSubcoreMesh(core_axis_name="c", subcore_axis_name="s"),
    scratch_types=(
        [
            pltpu.VMEM((NBUF, CB), jnp.int32),
            pltpu.VMEM((NBUF, CB), jnp.int32),
            pltpu.VMEM((NBUF, CB), jnp.int32),
            pltpu.VMEM((NBUF, CB), jnp.int32),
            pltpu.VMEM((NBUF, CB, D), jnp.float32),
        ]
        + [pltpu.SemaphoreType.DMA] * (3 * NBUF)
    ),
)(_body)


@jax.jit
def kernel(input, table):
    flat = input.reshape(B, 3)
    xs = flat[:, 0].reshape(B)
    ys = flat[:, 1].reshape(B)
    zs = flat[:, 2].reshape(B)
    return _gather(xs, ys, zs, table).reshape(BATCH, HIST, D)
